# baseline (speedup 1.0000x reference)
import functools

import jax
import jax.numpy as jnp
from jax import lax
from jax.experimental import pallas as pl
from jax.experimental.pallas import tpu as pltpu
from jax.experimental.pallas import tpu_sc as plsc

N_NODES = 10000
N_EDGES = 320000
HIDDEN = 128

NC, NS = 2, 16
NW = NC * NS
NPAD = 10240
CHUNK = 128
CPT = 80
EPAD = NW * CPT * CHUNK
RPT = NPAD // NS


_DEG_OUT = jax.ShapeDtypeStruct((NC * NPAD, HIDDEN), jnp.float32)
_DEG_SCRATCH = [
    pltpu.VMEM((CPT, CHUNK), jnp.int32),
    pltpu.VMEM((CHUNK, HIDDEN), jnp.float32),
    pltpu.VMEM_SHARED((NPAD, HIDDEN), jnp.float32),
    pltpu.SemaphoreType.DMA,
]


def _sc_deg_body(dst_hbm, ones_hbm, zeros_hbm, degp_hbm, didx_v, ones_v, deg_sh, sem):
    cid = lax.axis_index("c")
    sid = lax.axis_index("s")
    wid = sid * NC + cid
    pltpu.sync_copy(dst_hbm.at[pl.ds(wid * CPT, CPT)], didx_v)
    pltpu.sync_copy(ones_hbm, ones_v)
    pltpu.sync_copy(zeros_hbm.at[pl.ds(sid * RPT, RPT)], deg_sh.at[pl.ds(sid * RPT, RPT)])
    plsc.subcore_barrier()

    def fire(i, _):
        pltpu.async_copy(ones_v, deg_sh.at[didx_v.at[i]], sem, add=True)
        return 0

    lax.fori_loop(0, CPT, fire, 0)

    def drain(i, _):
        pltpu.make_async_copy(ones_v, deg_sh.at[didx_v.at[i]], sem).wait()
        return 0

    lax.fori_loop(0, CPT, drain, 0)
    plsc.subcore_barrier()
    pltpu.sync_copy(deg_sh.at[pl.ds(sid * RPT, RPT)],
                    degp_hbm.at[pl.ds(cid * NPAD + sid * RPT, RPT)])


IB = 16
NIB = CPT // IB

_SCAT_OUT = jax.ShapeDtypeStruct((NC * NPAD, HIDDEN), jnp.float32)
_SCAT_SCRATCH = [
    pltpu.VMEM((IB, CHUNK), jnp.int32),
    pltpu.VMEM((IB, CHUNK), jnp.int32),
    pltpu.VMEM((2, CHUNK, HIDDEN), jnp.float32),
    pltpu.VMEM_SHARED((NPAD, HIDDEN), jnp.float32),
    pltpu.SemaphoreType.DMA,
    pltpu.SemaphoreType.DMA,
]


def _sc_scatter_body(src_hbm, dst_hbm, y_hbm, zeros_hbm, accp_hbm,
                     sidx_v, didx_v, rows_v, acc_sh, sem0, sem1):
    cid = lax.axis_index("c")
    sid = lax.axis_index("s")
    wid = sid * NC + cid
    pltpu.sync_copy(zeros_hbm.at[pl.ds(sid * RPT, RPT)], acc_sh.at[pl.ds(sid * RPT, RPT)])
    plsc.subcore_barrier()

    sems_g = (sem0, sem1)

    def gather_start(ci, b):
        pltpu.async_copy(y_hbm.at[sidx_v.at[ci]], rows_v.at[b], sems_g[b])

    def gather_wait(ci, b):
        pltpu.make_async_copy(y_hbm.at[sidx_v.at[ci]], rows_v.at[b], sems_g[b]).wait()

    def blk(bi, _):
        boff = wid * CPT + bi * IB
        pltpu.sync_copy(src_hbm.at[pl.ds(boff, IB)], sidx_v)
        pltpu.sync_copy(dst_hbm.at[pl.ds(boff, IB)], didx_v)
        gather_start(0, 0)

        def body(g, _):
            c0 = g * 2
            gather_wait(c0, 0)
            gather_start(c0 + 1, 1)
            pltpu.sync_copy(rows_v.at[0], acc_sh.at[didx_v.at[c0]], add=True)
            gather_wait(c0 + 1, 1)

            @pl.when(c0 + 2 < IB)
            def _():
                gather_start(c0 + 2, 0)

            pltpu.sync_copy(rows_v.at[1], acc_sh.at[didx_v.at[c0 + 1]], add=True)
            return 0

        lax.fori_loop(0, IB // 2, body, 0)
        return 0

    lax.fori_loop(0, NIB, blk, 0)
    plsc.subcore_barrier()
    pltpu.sync_copy(acc_sh.at[pl.ds(sid * RPT, RPT)],
                    accp_hbm.at[pl.ds(cid * NPAD + sid * RPT, RPT)])


_mesh = plsc.VectorSubcoreMesh(core_axis_name="c", subcore_axis_name="s")
_sc_deg = pl.kernel(_sc_deg_body, out_type=_DEG_OUT, mesh=_mesh,
                    scratch_types=_DEG_SCRATCH)
_sc_scatter = pl.kernel(_sc_scatter_body, out_type=_SCAT_OUT, mesh=_mesh,
                        scratch_types=_SCAT_SCRATCH)


BN = 1024


def _dinv_block(dp0, dp1):
    deg = dp0[:, 0:1] + dp1[:, 0:1]
    return jnp.where(deg > 0, lax.rsqrt(jnp.maximum(deg, 1e-12)), 0.0)


def _elu(v):
    return jnp.where(v > 0, v, jnp.exp(v) - 1.0)


def _tc_prep_body(dp0, dp1, x_ref, w_ref, y_ref):
    dinv = _dinv_block(dp0[...], dp1[...])
    y_ref[...] = jnp.dot(x_ref[...], w_ref[...],
                         preferred_element_type=jnp.float32) * dinv


def _tc_mm_body(x_ref, w_ref, y_ref):
    y_ref[...] = jnp.dot(x_ref[...], w_ref[...],
                         preferred_element_type=jnp.float32)


def _tc_scale_body(xw_ref, dp0, dp1, y_ref):
    y_ref[...] = xw_ref[...] * _dinv_block(dp0[...], dp1[...])


def _tc_mid_body(ap0, ap1, dp0, dp1, b_ref, w_ref, y_ref):
    dinv = _dinv_block(dp0[...], dp1[...])
    h = _elu((ap0[...] + ap1[...]) * dinv + b_ref[...])
    y_ref[...] = jnp.dot(h, w_ref[...],
                         preferred_element_type=jnp.float32) * dinv


def _tc_final_body(ap0, ap1, dp0, dp1, b_ref, o_ref):
    dinv = _dinv_block(dp0[...], dp1[...])
    o_ref[...] = _elu((ap0[...] + ap1[...]) * dinv + b_ref[...])


_row_spec = pl.BlockSpec((BN, HIDDEN), lambda i: (i, 0))
_deg_spec = _row_spec
_w_spec = pl.BlockSpec((HIDDEN, HIDDEN), lambda i: (0, 0))
_b_spec = pl.BlockSpec((1, HIDDEN), lambda i: (0, 0))
_GRID = (NPAD // BN,)
_row_out = jax.ShapeDtypeStruct((NPAD, HIDDEN), jnp.float32)

_tc_prep = pl.pallas_call(
    _tc_prep_body, grid=_GRID,
    in_specs=[_deg_spec, _deg_spec, _row_spec, _w_spec],
    out_specs=_row_spec, out_shape=_row_out)

_tc_mm = pl.pallas_call(
    _tc_mm_body, grid=_GRID,
    in_specs=[_row_spec, _w_spec],
    out_specs=_row_spec, out_shape=_row_out)

_tc_scale = pl.pallas_call(
    _tc_scale_body, grid=_GRID,
    in_specs=[_row_spec, _deg_spec, _deg_spec],
    out_specs=_row_spec, out_shape=_row_out)

_tc_mid = pl.pallas_call(
    _tc_mid_body, grid=_GRID,
    in_specs=[_row_spec, _row_spec, _deg_spec, _deg_spec, _b_spec, _w_spec],
    out_specs=_row_spec, out_shape=_row_out)

_tc_final = pl.pallas_call(
    _tc_final_body, grid=_GRID,
    in_specs=[_row_spec, _row_spec, _deg_spec, _deg_spec, _b_spec],
    out_specs=_row_spec, out_shape=_row_out)


def kernel(x, edge_index, W1, b1, W2, b2):
    src = edge_index[0].astype(jnp.int32)
    dst = edge_index[1].astype(jnp.int32)
    pad = N_NODES + jnp.arange(EPAD - N_EDGES, dtype=jnp.int32) % (NPAD - N_NODES)
    srcp = jnp.concatenate([src, pad]).reshape(NW * CPT, CHUNK)
    dstp = jnp.concatenate([dst, pad]).reshape(NW * CPT, CHUNK)
    xp = jnp.pad(x, ((0, NPAD - N_NODES), (0, 0)))
    ones128 = jnp.ones((CHUNK, HIDDEN), jnp.float32)
    zeros128 = jnp.zeros((NPAD, HIDDEN), jnp.float32)

    xw = _tc_mm(xp, W1)
    degp = _sc_deg(dstp, ones128, zeros128)
    dp0, dp1 = degp[:NPAD], degp[NPAD:]

    y1 = _tc_scale(xw, dp0, dp1)
    accp = _sc_scatter(srcp, dstp, y1, zeros128)
    y2 = _tc_mid(accp[:NPAD], accp[NPAD:], dp0, dp1, b1.reshape(1, HIDDEN), W2)
    accp2 = _sc_scatter(srcp, dstp, y2, zeros128)
    out = _tc_final(accp2[:NPAD], accp2[NPAD:], dp0, dp1, b2.reshape(1, HIDDEN))
    return out[:N_NODES]


# speedup vs baseline: 1.0033x; 1.0033x over previous
import functools

import jax
import jax.numpy as jnp
from jax import lax
from jax.experimental import pallas as pl
from jax.experimental.pallas import tpu as pltpu
from jax.experimental.pallas import tpu_sc as plsc

N_NODES = 10000
N_EDGES = 320000
HIDDEN = 128

NC, NS = 2, 16
NW = NC * NS
NPAD = 10240
CHUNK = 128
CPT = 80
EPAD = NW * CPT * CHUNK
RPT = NPAD // NS


_DEG_OUT = jax.ShapeDtypeStruct((NC * NPAD, HIDDEN), jnp.float32)
_DEG_SCRATCH = [
    pltpu.VMEM((CPT, CHUNK), jnp.int32),
    pltpu.VMEM((CHUNK, HIDDEN), jnp.float32),
    pltpu.VMEM_SHARED((NPAD, HIDDEN), jnp.float32),
    pltpu.SemaphoreType.DMA,
]


def _sc_deg_body(dst_hbm, ones_hbm, zeros_hbm, degp_hbm, didx_v, ones_v, deg_sh, sem):
    cid = lax.axis_index("c")
    sid = lax.axis_index("s")
    wid = sid * NC + cid
    pltpu.sync_copy(dst_hbm.at[pl.ds(wid * CPT, CPT)], didx_v)
    pltpu.sync_copy(ones_hbm, ones_v)
    pltpu.sync_copy(zeros_hbm.at[pl.ds(sid * RPT, RPT)], deg_sh.at[pl.ds(sid * RPT, RPT)])
    plsc.subcore_barrier()

    def fire(i, _):
        pltpu.async_copy(ones_v, deg_sh.at[didx_v.at[i]], sem, add=True)
        return 0

    lax.fori_loop(0, CPT, fire, 0)

    def drain(i, _):
        pltpu.make_async_copy(ones_v, deg_sh.at[didx_v.at[i]], sem).wait()
        return 0

    lax.fori_loop(0, CPT, drain, 0)
    plsc.subcore_barrier()
    pltpu.sync_copy(deg_sh.at[pl.ds(sid * RPT, RPT)],
                    degp_hbm.at[pl.ds(cid * NPAD + sid * RPT, RPT)])


IB = 16
NIB = CPT // IB

_SCAT_OUT = jax.ShapeDtypeStruct((NC * NPAD, HIDDEN), jnp.float32)
_SCAT_SCRATCH = [
    pltpu.VMEM((IB, CHUNK), jnp.int32),
    pltpu.VMEM((IB, CHUNK), jnp.int32),
    pltpu.VMEM((2, CHUNK, HIDDEN), jnp.float32),
    pltpu.VMEM_SHARED((NPAD, HIDDEN), jnp.float32),
    pltpu.SemaphoreType.DMA,
    pltpu.SemaphoreType.DMA,
]


def _sc_scatter_body(src_hbm, dst_hbm, y_hbm, zeros_hbm, accp_hbm,
                     sidx_v, didx_v, rows_v, acc_sh, sem0, sem1):
    cid = lax.axis_index("c")
    sid = lax.axis_index("s")
    wid = sid * NC + cid
    pltpu.sync_copy(zeros_hbm.at[pl.ds(sid * RPT, RPT)], acc_sh.at[pl.ds(sid * RPT, RPT)])
    plsc.subcore_barrier()

    sems_g = (sem0, sem1)

    def gather_start(ci, b):
        pltpu.async_copy(y_hbm.at[sidx_v.at[ci]], rows_v.at[b], sems_g[b])

    def gather_wait(ci, b):
        pltpu.make_async_copy(y_hbm.at[sidx_v.at[ci]], rows_v.at[b], sems_g[b]).wait()

    def blk(bi, _):
        boff = wid * CPT + bi * IB
        pltpu.sync_copy(src_hbm.at[pl.ds(boff, IB)], sidx_v)
        pltpu.sync_copy(dst_hbm.at[pl.ds(boff, IB)], didx_v)
        gather_start(0, 0)

        def body(g, _):
            c0 = g * 2
            gather_wait(c0, 0)
            gather_start(c0 + 1, 1)
            pltpu.sync_copy(rows_v.at[0], acc_sh.at[didx_v.at[c0]], add=True)
            gather_wait(c0 + 1, 1)

            @pl.when(c0 + 2 < IB)
            def _():
                gather_start(c0 + 2, 0)

            pltpu.sync_copy(rows_v.at[1], acc_sh.at[didx_v.at[c0 + 1]], add=True)
            return 0

        lax.fori_loop(0, IB // 2, body, 0)
        return 0

    lax.fori_loop(0, NIB, blk, 0)
    plsc.subcore_barrier()
    pltpu.sync_copy(acc_sh.at[pl.ds(sid * RPT, RPT)],
                    accp_hbm.at[pl.ds(cid * NPAD + sid * RPT, RPT)])


_mesh = plsc.VectorSubcoreMesh(core_axis_name="c", subcore_axis_name="s")
_sc_deg = pl.kernel(_sc_deg_body, out_type=_DEG_OUT, mesh=_mesh,
                    scratch_types=_DEG_SCRATCH)
_sc_scatter = pl.kernel(_sc_scatter_body, out_type=_SCAT_OUT, mesh=_mesh,
                        scratch_types=_SCAT_SCRATCH)


BN = 1024


def _dinv_block(dp0, dp1):
    deg = dp0[:, 0:1] + dp1[:, 0:1]
    return jnp.where(deg > 0, lax.rsqrt(jnp.maximum(deg, 1e-12)), 0.0)


def _elu(v):
    return jnp.where(v > 0, v, jnp.exp(v) - 1.0)


def _tc_prep_body(dp0, dp1, x_ref, w_ref, y_ref):
    dinv = _dinv_block(dp0[...], dp1[...])
    y_ref[...] = jnp.dot(x_ref[...], w_ref[...],
                         preferred_element_type=jnp.float32) * dinv


def _tc_mm_body(x_ref, w_ref, y_ref):
    y_ref[...] = jnp.dot(x_ref[...], w_ref[...],
                         preferred_element_type=jnp.float32)


def _tc_scale_body(xw_ref, dp0, dp1, y_ref):
    y_ref[...] = xw_ref[...] * _dinv_block(dp0[...], dp1[...])


def _tc_mid_body(ap0, ap1, dp0, dp1, b_ref, w_ref, y_ref):
    dinv = _dinv_block(dp0[...], dp1[...])
    h = _elu((ap0[...] + ap1[...]) * dinv + b_ref[...])
    y_ref[...] = jnp.dot(h, w_ref[...],
                         preferred_element_type=jnp.float32) * dinv


def _tc_final_body(ap0, ap1, dp0, dp1, b_ref, o_ref):
    dinv = _dinv_block(dp0[...], dp1[...])
    o_ref[...] = _elu((ap0[...] + ap1[...]) * dinv + b_ref[...])


_row_spec = pl.BlockSpec((BN, HIDDEN), lambda i: (i, 0))
_deg_spec = _row_spec
_w_spec = pl.BlockSpec((HIDDEN, HIDDEN), lambda i: (0, 0))
_b_spec = pl.BlockSpec((1, HIDDEN), lambda i: (0, 0))
_GRID = (NPAD // BN,)
_row_out = jax.ShapeDtypeStruct((NPAD, HIDDEN), jnp.float32)

_tc_prep = pl.pallas_call(
    _tc_prep_body, grid=_GRID,
    in_specs=[_deg_spec, _deg_spec, _row_spec, _w_spec],
    out_specs=_row_spec, out_shape=_row_out)

_tc_mm = pl.pallas_call(
    _tc_mm_body, grid=_GRID,
    in_specs=[_row_spec, _w_spec],
    out_specs=_row_spec, out_shape=_row_out)

_tc_scale = pl.pallas_call(
    _tc_scale_body, grid=_GRID,
    in_specs=[_row_spec, _deg_spec, _deg_spec],
    out_specs=_row_spec, out_shape=_row_out)

_tc_mid = pl.pallas_call(
    _tc_mid_body, grid=_GRID,
    in_specs=[_row_spec, _row_spec, _deg_spec, _deg_spec, _b_spec, _w_spec],
    out_specs=_row_spec, out_shape=_row_out)

_tc_final = pl.pallas_call(
    _tc_final_body, grid=_GRID,
    in_specs=[_row_spec, _row_spec, _deg_spec, _deg_spec, _b_spec],
    out_specs=_row_spec, out_shape=_row_out)


def kernel(x, edge_index, W1, b1, W2, b2):
    src = edge_index[0].astype(jnp.int32)
    dst = edge_index[1].astype(jnp.int32)
    pad = N_NODES + jnp.arange(EPAD - N_EDGES, dtype=jnp.int32) % (NPAD - N_NODES)
    srcp = jnp.concatenate([src, pad]).reshape(NW * CPT, CHUNK)
    dstp = jnp.concatenate([dst, pad]).reshape(NW * CPT, CHUNK)
    xp = jnp.pad(x, ((0, NPAD - N_NODES), (0, 0)))
    ones128 = jnp.ones((CHUNK, HIDDEN), jnp.float32)
    zeros128 = jnp.zeros((NPAD, HIDDEN), jnp.float32)

    degp = _sc_deg(dstp, ones128, zeros128)
    dp0, dp1 = degp[:NPAD], degp[NPAD:]

    y1 = _tc_prep(dp0, dp1, xp, W1)
    accp = _sc_scatter(srcp, dstp, y1, zeros128)
    y2 = _tc_mid(accp[:NPAD], accp[NPAD:], dp0, dp1, b1.reshape(1, HIDDEN), W2)
    accp2 = _sc_scatter(srcp, dstp, y2, zeros128)
    out = _tc_final(accp2[:NPAD], accp2[NPAD:], dp0, dp1, b2.reshape(1, HIDDEN))
    return out[:N_NODES]
